# baseline (speedup 1.0000x reference)
import functools

import jax
import jax.numpy as jnp
from jax import lax
from jax.experimental import pallas as pl
from jax.experimental.pallas import tpu as pltpu
from jax.experimental.pallas import tpu_sc as plsc

R = 131072
DEPTH = 8
DIM = 8
NPTS = 4
TABLE_SIZE = 4194304
NGRP = TABLE_SIZE // 128

_PIS = (774363409, 2654435761, 805459861, 100000007,
        334363391, 1334363413, 734363407, 2134363393)
K_XOR = tuple(((c + 1) * _PIS[c]) % TABLE_SIZE for c in range(8))

NC = 2
NS = 16
NW = NC * NS
RPW = R // NW
B_R = 128
NBLK = RPW // B_R
NG = B_R // 16
GPW = NGRP // NW
GB = 16
CW = GB * 1024
CPW = GPW // GB

_SC_PARAMS = dict(
    compiler_params=pltpu.CompilerParams(use_tc_tiling_on_sc=False,
                                         needs_layout_passes=False),
)


def _splat(v):
    return jnp.broadcast_to(jnp.asarray(v, jnp.int32), (16,))


def _fmt_body(tsrc_h, tdst_h, tin_v, tout_v, insem, outsem):
    wid = lax.axis_index("s") * NC + lax.axis_index("c")
    iota = lax.iota(jnp.int32, 16)
    rowcol0 = (iota & 7) * 128 + (iota >> 3)
    base = wid * (GPW * 1024)

    def fire_in(ci, slot):
        pltpu.async_copy(tsrc_h.at[pl.ds(base + ci * CW, CW)],
                         tin_v.at[jnp.int32(slot)], insem.at[jnp.int32(slot)])

    def drain_in(slot):
        pltpu.make_async_copy(tsrc_h.at[pl.ds(base, CW)],
                              tin_v.at[jnp.int32(slot)],
                              insem.at[jnp.int32(slot)]).wait()

    def fire_out(ci, slot):
        pltpu.async_copy(tout_v.at[jnp.int32(slot)],
                         tdst_h.at[pl.ds(base + ci * CW, CW)],
                         outsem.at[jnp.int32(slot)])

    def drain_out(slot):
        pltpu.make_async_copy(tout_v.at[jnp.int32(slot)],
                              tdst_h.at[pl.ds(base, CW)],
                              outsem.at[jnp.int32(slot)]).wait()

    def process(slot):
        @pl.loop(jnp.int32(0), jnp.int32(GB * 64))
        def t_body(g2):
            grp = g2 >> 6
            c2 = g2 & 63
            idx = rowcol0 + (grp * 1024 + 2 * c2)
            vals = plsc.load_gather(tin_v, [_splat(slot), idx])
            tout_v[jnp.int32(slot), pl.ds(g2 * 16, 16)] = vals

    fire_in(jnp.int32(0), 0)

    @pl.loop(jnp.int32(0), jnp.int32(CPW), step=2)
    def chunk_body(ci):
        fire_in(ci + 1, 1)

        @pl.when(ci > 0)
        def _():
            drain_out(0)
        drain_in(0)
        process(0)
        fire_out(ci, 0)

        @pl.when(ci + 2 < CPW)
        def _():
            fire_in(ci + 2, 0)

        @pl.when(ci > 0)
        def _():
            drain_out(1)
        drain_in(1)
        process(1)
        fire_out(ci + 1, 1)

    drain_out(0)
    drain_out(1)


@functools.cache
def _build_fmt_kernel():
    return pl.kernel(
        _fmt_body,
        out_type=jax.ShapeDtypeStruct((TABLE_SIZE * DIM,), jnp.float32),
        mesh=plsc.VectorSubcoreMesh(core_axis_name="c", subcore_axis_name="s",
                                    num_cores=NC, num_subcores=NS),
        scratch_types=[
            pltpu.VMEM((2, CW), jnp.float32),
            pltpu.VMEM((2, CW), jnp.float32),
            pltpu.SemaphoreType.DMA((2,)),
            pltpu.SemaphoreType.DMA((2,)),
        ],
        **_SC_PARAMS,
    )


def _sc_body(table_h, hist_h, inp_h, nodes_h, out_h,
             jall_v, j_v, tidx_v, node_v, feat_v, inp_v, out_v, gsem):
    wid = lax.axis_index("s") * NC + lax.axis_index("c")
    iota = lax.iota(jnp.int32, 16)
    laneconst = (iota >> 3) * 2048 + (iota & 7) * 128

    def build_and_fire(i, slot):
        @pl.loop(jnp.int32(0), jnp.int32(NG))
        def tidx_body(g):
            lane = g * 16 + iota
            jv = plsc.load_gather(jall_v, [_splat(i), lane])
            j_v[jnp.int32(slot), pl.ds(g * 16, 16)] = jv
            for c in range(8):
                tidx_v[jnp.int32(slot), jnp.int32(c), pl.ds(g * 16, 16)] = (
                    jv ^ jnp.int32(K_XOR[c]))

        pltpu.async_copy(nodes_h.at[j_v.at[jnp.int32(slot)]],
                         node_v.at[jnp.int32(slot)], gsem.at[jnp.int32(slot)])
        for c in range(8):
            pltpu.async_copy(
                table_h.at[tidx_v.at[jnp.int32(slot), jnp.int32(c)]],
                feat_v.at[jnp.int32(slot), jnp.int32(c)],
                gsem.at[jnp.int32(slot)])

    def drain(slot):
        pltpu.make_async_copy(nodes_h.at[j_v.at[jnp.int32(slot)]],
                              node_v.at[jnp.int32(slot)],
                              gsem.at[jnp.int32(slot)]).wait()
        for c in range(8):
            pltpu.make_async_copy(
                table_h.at[tidx_v.at[jnp.int32(slot), jnp.int32(c)]],
                feat_v.at[jnp.int32(slot), jnp.int32(c)],
                gsem.at[jnp.int32(slot)]).wait()

    def compute(i, slot):
        ihi = i >> 2
        ilo = i & 3
        colbase = ihi * 1024 + ilo * 32

        @pl.loop(jnp.int32(0), jnp.int32(NG))
        def group_body(g):
            lane = g * 16 + iota
            laneterm = laneconst + g * 4096
            nm = [plsc.load_gather(node_v, [_splat(slot), lane, _splat(comp)])
                  for comp in range(3)]
            ie = [1.0 / plsc.load_gather(node_v,
                                         [_splat(slot), lane, _splat(3 + comp)])
                  for comp in range(3)]
            p = [[jnp.clip(
                (inp_v[jnp.int32(3 * k + comp), pl.ds(g * 16, 16)]
                 - nm[comp]) * ie[comp], 0.0, 1.0) for comp in range(3)]
                for k in range(NPTS)]
            for d in range(DIM):
                f = [plsc.load_gather(
                    feat_v, [_splat(slot), _splat(c), lane, _splat(d)])
                    for c in range(8)]
                d00 = f[3] - f[0]
                d10 = f[4] - f[1]
                d01 = f[5] - f[2]
                d11 = f[7] - f[6]
                for k in range(NPTS):
                    px, py, pz = p[k]
                    t00 = f[0] + pz * d00
                    t10 = f[1] + pz * d10
                    t01 = f[2] + pz * d01
                    t11 = f[6] + pz * d11
                    ty0 = t00 + py * (t01 - t00)
                    ty1 = t10 + py * (t11 - t10)
                    acc = ty0 + px * (ty1 - ty0)
                    pos = laneterm + (colbase + (k * 8 + d))
                    plsc.store_scatter(out_v, [pos], acc)

    @pl.loop(jnp.int32(0), jnp.int32(NBLK))
    def block_body(b):
        base = wid * RPW + b * B_R
        for row in range(DEPTH):
            pltpu.sync_copy(hist_h.at[jnp.int32(row), pl.ds(base, B_R)],
                            jall_v.at[jnp.int32(row)])
        for row in range(3 * NPTS):
            pltpu.sync_copy(inp_h.at[jnp.int32(row), pl.ds(base, B_R)],
                            inp_v.at[jnp.int32(row)])

        build_and_fire(jnp.int32(0), 0)

        @pl.loop(jnp.int32(0), jnp.int32(DEPTH), step=2)
        def depth_body(i):
            build_and_fire(i + 1, 1)
            drain(0)
            compute(i, 0)

            @pl.when(i + 2 < DEPTH)
            def _():
                build_and_fire(i + 2, 0)

            drain(1)
            compute(i + 1, 1)

        pltpu.sync_copy(out_v, out_h.at[pl.ds(base * 256, B_R * 256)])


@functools.cache
def _build_sc_kernel():
    return pl.kernel(
        _sc_body,
        out_type=jax.ShapeDtypeStruct((R * DEPTH * NPTS * DIM,), jnp.float32),
        mesh=plsc.VectorSubcoreMesh(core_axis_name="c", subcore_axis_name="s",
                                    num_cores=NC, num_subcores=NS),
        scratch_types=[
            pltpu.VMEM((DEPTH, B_R), jnp.int32),
            pltpu.VMEM((2, B_R), jnp.int32),
            pltpu.VMEM((2, 8, B_R), jnp.int32),
            pltpu.VMEM((2, B_R, 8), jnp.float32),
            pltpu.VMEM((2, 8, B_R, DIM), jnp.float32),
            pltpu.VMEM((3 * NPTS, B_R), jnp.float32),
            pltpu.VMEM((B_R * DEPTH * NPTS * DIM,), jnp.float32),
            pltpu.SemaphoreType.DMA((2,)),
        ],
        **_SC_PARAMS,
    )


def kernel(inp, history, table, nodes_min, nodes_extent):
    table_t = (table.reshape(NGRP, 128, DIM).transpose(0, 2, 1)
               .reshape(NGRP * DIM * 128))
    table_lin = _build_fmt_kernel()(table_t).reshape(TABLE_SIZE, DIM)

    hist_t = history.astype(jnp.int32).T
    inp_t = inp.astype(jnp.float32).reshape(R, 3 * NPTS).T
    nodes_cat = jnp.concatenate(
        [nodes_min.astype(jnp.float32), nodes_extent.astype(jnp.float32),
         jnp.zeros((nodes_min.shape[0], 2), jnp.float32)], axis=1)
    raw = _build_sc_kernel()(table_lin, hist_t, inp_t, nodes_cat)
    return (raw.reshape(R // 8, 2, 8, 128).transpose(0, 2, 1, 3)
            .reshape(R, DEPTH * NPTS * DIM))


# speedup vs baseline: 1.1600x; 1.1600x over previous
import functools

import jax
import jax.numpy as jnp
from jax import lax
from jax.experimental import pallas as pl
from jax.experimental.pallas import tpu as pltpu
from jax.experimental.pallas import tpu_sc as plsc

R = 131072
DEPTH = 8
DIM = 8
NPTS = 4
TABLE_SIZE = 4194304
NGRP = TABLE_SIZE // 128

_PIS = (774363409, 2654435761, 805459861, 100000007,
        334363391, 1334363413, 734363407, 2134363393)
K_XOR = tuple(((c + 1) * _PIS[c]) % TABLE_SIZE for c in range(8))

NC = 2
NS = 16
NW = NC * NS
RPW = R // NW
B_R = 128
NBLK = RPW // B_R
NG = B_R // 16
GPW = NGRP // NW
GB = 16
CW = GB * 1024
CPW = GPW // GB

_SC_PARAMS = dict(
    compiler_params=pltpu.CompilerParams(use_tc_tiling_on_sc=False,
                                         needs_layout_passes=False),
)


def _splat(v):
    return jnp.broadcast_to(jnp.asarray(v, jnp.int32), (16,))


def _fmt_body(tsrc_h, tdst_h, tin_v, tout_v, insem, outsem):
    wid = lax.axis_index("s") * NC + lax.axis_index("c")
    iota = lax.iota(jnp.int32, 16)
    rowcol0 = (iota & 7) * 128 + (iota >> 3)
    base = wid * (GPW * 1024)

    def fire_in(ci, slot):
        pltpu.async_copy(tsrc_h.at[pl.ds(base + ci * CW, CW)],
                         tin_v.at[jnp.int32(slot)], insem.at[jnp.int32(slot)])

    def drain_in(slot):
        pltpu.make_async_copy(tsrc_h.at[pl.ds(base, CW)],
                              tin_v.at[jnp.int32(slot)],
                              insem.at[jnp.int32(slot)]).wait()

    def fire_out(ci, slot):
        pltpu.async_copy(tout_v.at[jnp.int32(slot)],
                         tdst_h.at[pl.ds(base + ci * CW, CW)],
                         outsem.at[jnp.int32(slot)])

    def drain_out(slot):
        pltpu.make_async_copy(tout_v.at[jnp.int32(slot)],
                              tdst_h.at[pl.ds(base, CW)],
                              outsem.at[jnp.int32(slot)]).wait()

    def process(slot):
        @pl.loop(jnp.int32(0), jnp.int32(GB * 64))
        def t_body(g2):
            grp = g2 >> 6
            c2 = g2 & 63
            idx = rowcol0 + (grp * 1024 + 2 * c2)
            vals = plsc.load_gather(tin_v, [_splat(slot), idx])
            tout_v[jnp.int32(slot), pl.ds(g2 * 16, 16)] = vals

    fire_in(jnp.int32(0), 0)

    @pl.loop(jnp.int32(0), jnp.int32(CPW), step=2)
    def chunk_body(ci):
        fire_in(ci + 1, 1)

        @pl.when(ci > 0)
        def _():
            drain_out(0)
        drain_in(0)
        process(0)
        fire_out(ci, 0)

        @pl.when(ci + 2 < CPW)
        def _():
            fire_in(ci + 2, 0)

        @pl.when(ci > 0)
        def _():
            drain_out(1)
        drain_in(1)
        process(1)
        fire_out(ci + 1, 1)

    drain_out(0)
    drain_out(1)


@functools.cache
def _build_fmt_kernel():
    return pl.kernel(
        _fmt_body,
        out_type=jax.ShapeDtypeStruct((TABLE_SIZE * DIM,), jnp.float32),
        mesh=plsc.VectorSubcoreMesh(core_axis_name="c", subcore_axis_name="s",
                                    num_cores=NC, num_subcores=NS),
        scratch_types=[
            pltpu.VMEM((2, CW), jnp.float32),
            pltpu.VMEM((2, CW), jnp.float32),
            pltpu.SemaphoreType.DMA((2,)),
            pltpu.SemaphoreType.DMA((2,)),
        ],
        **_SC_PARAMS,
    )


def _sc_body(table_h, hist_h, inp_h, nodes_h, out_h,
             jall_v, j_v, tidx_v, node_v, feat_v, inp_v, out_v, gsem):
    wid = lax.axis_index("s") * NC + lax.axis_index("c")
    iota = lax.iota(jnp.int32, 16)
    laneconst = (iota >> 3) * 2048 + (iota & 7) * 128

    def build_and_fire(i, slot):
        @pl.loop(jnp.int32(0), jnp.int32(NG))
        def tidx_body(g):
            lane = g * 16 + iota
            jv = plsc.load_gather(jall_v, [_splat(i), lane])
            j_v[jnp.int32(slot), pl.ds(g * 16, 16)] = jv
            for c in range(8):
                tidx_v[jnp.int32(slot), pl.ds(c * B_R + g * 16, 16)] = (
                    jv ^ jnp.int32(K_XOR[c]))

        pltpu.async_copy(nodes_h.at[j_v.at[jnp.int32(slot)]],
                         node_v.at[jnp.int32(slot)], gsem.at[jnp.int32(slot)])
        pltpu.async_copy(table_h.at[tidx_v.at[jnp.int32(slot)]],
                         feat_v.at[jnp.int32(slot)], gsem.at[jnp.int32(slot)])

    def drain(slot):
        pltpu.make_async_copy(nodes_h.at[j_v.at[jnp.int32(slot)]],
                              node_v.at[jnp.int32(slot)],
                              gsem.at[jnp.int32(slot)]).wait()
        pltpu.make_async_copy(table_h.at[tidx_v.at[jnp.int32(slot)]],
                              feat_v.at[jnp.int32(slot)],
                              gsem.at[jnp.int32(slot)]).wait()

    def compute(i, slot):
        ihi = i >> 2
        ilo = i & 3
        colbase = ihi * 1024 + ilo * 32

        @pl.loop(jnp.int32(0), jnp.int32(NG))
        def group_body(g):
            lane = g * 16 + iota
            laneterm = laneconst + g * 4096
            nm = [plsc.load_gather(node_v, [_splat(slot), lane, _splat(comp)])
                  for comp in range(3)]
            ie = [1.0 / plsc.load_gather(node_v,
                                         [_splat(slot), lane, _splat(3 + comp)])
                  for comp in range(3)]
            w = []
            for k in range(NPTS):
                px = jnp.clip((inp_v[jnp.int32(3 * k + 0), pl.ds(g * 16, 16)] - nm[0]) * ie[0], 0.0, 1.0)
                py = jnp.clip((inp_v[jnp.int32(3 * k + 1), pl.ds(g * 16, 16)] - nm[1]) * ie[1], 0.0, 1.0)
                pz = jnp.clip((inp_v[jnp.int32(3 * k + 2), pl.ds(g * 16, 16)] - nm[2]) * ie[2], 0.0, 1.0)
                ax, ay, az = 1.0 - px, 1.0 - py, 1.0 - pz
                b00, b10, b01, b11 = ax * ay, px * ay, ax * py, px * py
                w.append((b00 * az, b10 * az, b01 * az, b00 * pz,
                          b10 * pz, b01 * pz, b11 * az, b11 * pz))
            for d in range(DIM):
                f = [plsc.load_gather(
                    feat_v, [_splat(slot), _splat(c * B_R) + lane, _splat(d)])
                    for c in range(8)]
                for k in range(NPTS):
                    acc = w[k][0] * f[0]
                    for c in range(1, 8):
                        acc = acc + w[k][c] * f[c]
                    pos = laneterm + (colbase + (k * 8 + d))
                    plsc.store_scatter(out_v, [pos], acc)

    @pl.loop(jnp.int32(0), jnp.int32(NBLK))
    def block_body(b):
        base = wid * RPW + b * B_R
        pltpu.sync_copy(hist_h.at[:, pl.ds(base, B_R)], jall_v)
        pltpu.sync_copy(inp_h.at[:, pl.ds(base, B_R)], inp_v)

        build_and_fire(jnp.int32(0), 0)

        @pl.loop(jnp.int32(0), jnp.int32(DEPTH), step=2)
        def depth_body(i):
            build_and_fire(i + 1, 1)
            drain(0)
            compute(i, 0)

            @pl.when(i + 2 < DEPTH)
            def _():
                build_and_fire(i + 2, 0)

            drain(1)
            compute(i + 1, 1)

        pltpu.sync_copy(out_v, out_h.at[pl.ds(base * 256, B_R * 256)])


@functools.cache
def _build_sc_kernel():
    return pl.kernel(
        _sc_body,
        out_type=jax.ShapeDtypeStruct((R * DEPTH * NPTS * DIM,), jnp.float32),
        mesh=plsc.VectorSubcoreMesh(core_axis_name="c", subcore_axis_name="s",
                                    num_cores=NC, num_subcores=NS),
        scratch_types=[
            pltpu.VMEM((DEPTH, B_R), jnp.int32),
            pltpu.VMEM((2, B_R), jnp.int32),
            pltpu.VMEM((2, 8 * B_R), jnp.int32),
            pltpu.VMEM((2, B_R, 8), jnp.float32),
            pltpu.VMEM((2, 8 * B_R, DIM), jnp.float32),
            pltpu.VMEM((3 * NPTS, B_R), jnp.float32),
            pltpu.VMEM((B_R * DEPTH * NPTS * DIM,), jnp.float32),
            pltpu.SemaphoreType.DMA((2,)),
        ],
        **_SC_PARAMS,
    )


def kernel(inp, history, table, nodes_min, nodes_extent):
    table_t = (table.reshape(NGRP, 128, DIM).transpose(0, 2, 1)
               .reshape(NGRP * DIM * 128))
    table_lin = _build_fmt_kernel()(table_t).reshape(TABLE_SIZE, DIM)

    hist_t = history.astype(jnp.int32).T
    inp_t = inp.astype(jnp.float32).reshape(R, 3 * NPTS).T
    nodes_cat = jnp.concatenate(
        [nodes_min.astype(jnp.float32), nodes_extent.astype(jnp.float32),
         jnp.zeros((nodes_min.shape[0], 2), jnp.float32)], axis=1)
    raw = _build_sc_kernel()(table_lin, hist_t, inp_t, nodes_cat)
    return (raw.reshape(R // 8, 2, 8, 128).transpose(0, 2, 1, 3)
            .reshape(R, DEPTH * NPTS * DIM))


# speedup vs baseline: 1.1731x; 1.0113x over previous
import functools

import jax
import jax.numpy as jnp
from jax import lax
from jax.experimental import pallas as pl
from jax.experimental.pallas import tpu as pltpu
from jax.experimental.pallas import tpu_sc as plsc

R = 131072
DEPTH = 8
DIM = 8
NPTS = 4
TABLE_SIZE = 4194304
NGRP = TABLE_SIZE // 128

_PIS = (774363409, 2654435761, 805459861, 100000007,
        334363391, 1334363413, 734363407, 2134363393)
K_XOR = tuple(((c + 1) * _PIS[c]) % TABLE_SIZE for c in range(8))

NC = 2
NS = 16
NW = NC * NS
RPW = R // NW
B_R = 256
NBLK = RPW // B_R
NG = B_R // 16
GPW = NGRP // NW
GB = 16
CW = GB * 1024
CPW = GPW // GB

_SC_PARAMS = dict(
    compiler_params=pltpu.CompilerParams(use_tc_tiling_on_sc=False,
                                         needs_layout_passes=False),
)


def _splat(v):
    return jnp.broadcast_to(jnp.asarray(v, jnp.int32), (16,))


def _fmt_body(tsrc_h, tdst_h, tin_v, tout_v, insem, outsem):
    wid = lax.axis_index("s") * NC + lax.axis_index("c")
    iota = lax.iota(jnp.int32, 16)
    rowcol0 = (iota & 7) * 128 + (iota >> 3)
    base = wid * (GPW * 1024)

    def fire_in(ci, slot):
        pltpu.async_copy(tsrc_h.at[pl.ds(base + ci * CW, CW)],
                         tin_v.at[jnp.int32(slot)], insem.at[jnp.int32(slot)])

    def drain_in(slot):
        pltpu.make_async_copy(tsrc_h.at[pl.ds(base, CW)],
                              tin_v.at[jnp.int32(slot)],
                              insem.at[jnp.int32(slot)]).wait()

    def fire_out(ci, slot):
        pltpu.async_copy(tout_v.at[jnp.int32(slot)],
                         tdst_h.at[pl.ds(base + ci * CW, CW)],
                         outsem.at[jnp.int32(slot)])

    def drain_out(slot):
        pltpu.make_async_copy(tout_v.at[jnp.int32(slot)],
                              tdst_h.at[pl.ds(base, CW)],
                              outsem.at[jnp.int32(slot)]).wait()

    def process(slot):
        @pl.loop(jnp.int32(0), jnp.int32(GB * 16))
        def t_body(q):
            for u in range(4):
                g2 = q * 4 + u
                grp = g2 >> 6
                c2 = g2 & 63
                idx = rowcol0 + (grp * 1024 + 2 * c2)
                vals = plsc.load_gather(tin_v, [_splat(slot), idx])
                tout_v[jnp.int32(slot), pl.ds(g2 * 16, 16)] = vals

    fire_in(jnp.int32(0), 0)

    @pl.loop(jnp.int32(0), jnp.int32(CPW), step=2)
    def chunk_body(ci):
        fire_in(ci + 1, 1)

        @pl.when(ci > 0)
        def _():
            drain_out(0)
        drain_in(0)
        process(0)
        fire_out(ci, 0)

        @pl.when(ci + 2 < CPW)
        def _():
            fire_in(ci + 2, 0)

        @pl.when(ci > 0)
        def _():
            drain_out(1)
        drain_in(1)
        process(1)
        fire_out(ci + 1, 1)

    drain_out(0)
    drain_out(1)


@functools.cache
def _build_fmt_kernel():
    return pl.kernel(
        _fmt_body,
        out_type=jax.ShapeDtypeStruct((TABLE_SIZE * DIM,), jnp.float32),
        mesh=plsc.VectorSubcoreMesh(core_axis_name="c", subcore_axis_name="s",
                                    num_cores=NC, num_subcores=NS),
        scratch_types=[
            pltpu.VMEM((2, CW), jnp.float32),
            pltpu.VMEM((2, CW), jnp.float32),
            pltpu.SemaphoreType.DMA((2,)),
            pltpu.SemaphoreType.DMA((2,)),
        ],
        **_SC_PARAMS,
    )


def _sc_body(table_h, hist_h, inp_h, nodes_h, out_h,
             jall_v, j_v, tidx_v, node_v, feat_v, inp_v, out_v, gsem):
    wid = lax.axis_index("s") * NC + lax.axis_index("c")
    iota = lax.iota(jnp.int32, 16)
    laneconst = (iota >> 3) * 2048 + (iota & 7) * 128

    def build_and_fire(i, slot):
        @pl.loop(jnp.int32(0), jnp.int32(NG))
        def tidx_body(g):
            lane = g * 16 + iota
            jv = plsc.load_gather(jall_v, [_splat(i), lane])
            j_v[jnp.int32(slot), pl.ds(g * 16, 16)] = jv
            for c in range(8):
                tidx_v[jnp.int32(slot), pl.ds(c * B_R + g * 16, 16)] = (
                    jv ^ jnp.int32(K_XOR[c]))

        pltpu.async_copy(nodes_h.at[j_v.at[jnp.int32(slot)]],
                         node_v.at[jnp.int32(slot)], gsem.at[jnp.int32(slot)])
        pltpu.async_copy(table_h.at[tidx_v.at[jnp.int32(slot)]],
                         feat_v.at[jnp.int32(slot)], gsem.at[jnp.int32(slot)])

    def drain(slot):
        pltpu.make_async_copy(nodes_h.at[j_v.at[jnp.int32(slot)]],
                              node_v.at[jnp.int32(slot)],
                              gsem.at[jnp.int32(slot)]).wait()
        pltpu.make_async_copy(table_h.at[tidx_v.at[jnp.int32(slot)]],
                              feat_v.at[jnp.int32(slot)],
                              gsem.at[jnp.int32(slot)]).wait()

    def compute(i, slot):
        ihi = i >> 2
        ilo = i & 3
        colbase = ihi * 1024 + ilo * 32

        @pl.loop(jnp.int32(0), jnp.int32(NG))
        def group_body(g):
            lane = g * 16 + iota
            laneterm = laneconst + g * 4096
            nm = [plsc.load_gather(node_v, [_splat(slot), lane, _splat(comp)])
                  for comp in range(3)]
            ie = [1.0 / plsc.load_gather(node_v,
                                         [_splat(slot), lane, _splat(3 + comp)])
                  for comp in range(3)]
            w = []
            for k in range(NPTS):
                px = jnp.clip((inp_v[jnp.int32(3 * k + 0), pl.ds(g * 16, 16)] - nm[0]) * ie[0], 0.0, 1.0)
                py = jnp.clip((inp_v[jnp.int32(3 * k + 1), pl.ds(g * 16, 16)] - nm[1]) * ie[1], 0.0, 1.0)
                pz = jnp.clip((inp_v[jnp.int32(3 * k + 2), pl.ds(g * 16, 16)] - nm[2]) * ie[2], 0.0, 1.0)
                ax, ay, az = 1.0 - px, 1.0 - py, 1.0 - pz
                b00, b10, b01, b11 = ax * ay, px * ay, ax * py, px * py
                w.append((b00 * az, b10 * az, b01 * az, b00 * pz,
                          b10 * pz, b01 * pz, b11 * az, b11 * pz))
            for d in range(DIM):
                f = [plsc.load_gather(
                    feat_v, [_splat(slot), _splat(c * B_R) + lane, _splat(d)])
                    for c in range(8)]
                for k in range(NPTS):
                    acc = w[k][0] * f[0]
                    for c in range(1, 8):
                        acc = acc + w[k][c] * f[c]
                    pos = laneterm + (colbase + (k * 8 + d))
                    plsc.store_scatter(out_v, [pos], acc)

    @pl.loop(jnp.int32(0), jnp.int32(NBLK))
    def block_body(b):
        base = wid * RPW + b * B_R
        pltpu.sync_copy(hist_h.at[:, pl.ds(base, B_R)], jall_v)
        pltpu.sync_copy(inp_h.at[:, pl.ds(base, B_R)], inp_v)

        build_and_fire(jnp.int32(0), 0)

        @pl.loop(jnp.int32(0), jnp.int32(DEPTH), step=2)
        def depth_body(i):
            build_and_fire(i + 1, 1)
            drain(0)
            compute(i, 0)

            @pl.when(i + 2 < DEPTH)
            def _():
                build_and_fire(i + 2, 0)

            drain(1)
            compute(i + 1, 1)

        pltpu.sync_copy(out_v, out_h.at[pl.ds(base * 256, B_R * 256)])


@functools.cache
def _build_sc_kernel():
    return pl.kernel(
        _sc_body,
        out_type=jax.ShapeDtypeStruct((R * DEPTH * NPTS * DIM,), jnp.float32),
        mesh=plsc.VectorSubcoreMesh(core_axis_name="c", subcore_axis_name="s",
                                    num_cores=NC, num_subcores=NS),
        scratch_types=[
            pltpu.VMEM((DEPTH, B_R), jnp.int32),
            pltpu.VMEM((2, B_R), jnp.int32),
            pltpu.VMEM((2, 8 * B_R), jnp.int32),
            pltpu.VMEM((2, B_R, 8), jnp.float32),
            pltpu.VMEM((2, 8 * B_R, DIM), jnp.float32),
            pltpu.VMEM((3 * NPTS, B_R), jnp.float32),
            pltpu.VMEM((B_R * DEPTH * NPTS * DIM,), jnp.float32),
            pltpu.SemaphoreType.DMA((2,)),
        ],
        **_SC_PARAMS,
    )


def kernel(inp, history, table, nodes_min, nodes_extent):
    table_t = (table.reshape(NGRP, 128, DIM).transpose(0, 2, 1)
               .reshape(NGRP * DIM * 128))
    table_lin = _build_fmt_kernel()(table_t).reshape(TABLE_SIZE, DIM)

    hist_t = history.astype(jnp.int32).T
    inp_t = inp.astype(jnp.float32).reshape(R, 3 * NPTS).T
    nodes_cat = jnp.concatenate(
        [nodes_min.astype(jnp.float32), nodes_extent.astype(jnp.float32),
         jnp.zeros((nodes_min.shape[0], 2), jnp.float32)], axis=1)
    raw = _build_sc_kernel()(table_lin, hist_t, inp_t, nodes_cat)
    return (raw.reshape(R // 8, 2, 8, 128).transpose(0, 2, 1, 3)
            .reshape(R, DEPTH * NPTS * DIM))


# speedup vs baseline: 1.1792x; 1.0052x over previous
import functools

import jax
import jax.numpy as jnp
from jax import lax
from jax.experimental import pallas as pl
from jax.experimental.pallas import tpu as pltpu
from jax.experimental.pallas import tpu_sc as plsc

R = 131072
DEPTH = 8
DIM = 8
NPTS = 4
TABLE_SIZE = 4194304
NGRP = TABLE_SIZE // 128

_PIS = (774363409, 2654435761, 805459861, 100000007,
        334363391, 1334363413, 734363407, 2134363393)
K_XOR = tuple(((c + 1) * _PIS[c]) % TABLE_SIZE for c in range(8))

NC = 2
NS = 16
NW = NC * NS
RPW = R // NW
B_R = 256
NBLK = RPW // B_R
NG = B_R // 16
GPW = NGRP // NW
GB = 16
CW = GB * 1024
CPW = GPW // GB

_SC_PARAMS = dict(
    compiler_params=pltpu.CompilerParams(use_tc_tiling_on_sc=False,
                                         needs_layout_passes=False),
)


def _splat(v):
    return jnp.broadcast_to(jnp.asarray(v, jnp.int32), (16,))


def _fmt_body(tsrc_h, tdst_h, tin_v, tout_v, insem, outsem):
    wid = lax.axis_index("s") * NC + lax.axis_index("c")
    iota = lax.iota(jnp.int32, 16)
    rowcol0 = (iota & 7) * 128 + (iota >> 3)
    base = wid * (GPW * 1024)

    def fire_in(ci, slot):
        pltpu.async_copy(tsrc_h.at[pl.ds(base + ci * CW, CW)],
                         tin_v.at[jnp.int32(slot)], insem.at[jnp.int32(slot)])

    def drain_in(slot):
        pltpu.make_async_copy(tsrc_h.at[pl.ds(base, CW)],
                              tin_v.at[jnp.int32(slot)],
                              insem.at[jnp.int32(slot)]).wait()

    def fire_out(ci, slot):
        pltpu.async_copy(tout_v.at[jnp.int32(slot)],
                         tdst_h.at[pl.ds(base + ci * CW, CW)],
                         outsem.at[jnp.int32(slot)])

    def drain_out(slot):
        pltpu.make_async_copy(tout_v.at[jnp.int32(slot)],
                              tdst_h.at[pl.ds(base, CW)],
                              outsem.at[jnp.int32(slot)]).wait()

    def process(slot):
        @pl.loop(jnp.int32(0), jnp.int32(GB * 8))
        def t_body(q):
            for u in range(8):
                g2 = q * 8 + u
                grp = g2 >> 6
                c2 = g2 & 63
                idx = rowcol0 + (grp * 1024 + 2 * c2)
                vals = plsc.load_gather(tin_v, [_splat(slot), idx])
                tout_v[jnp.int32(slot), pl.ds(g2 * 16, 16)] = vals

    fire_in(jnp.int32(0), 0)

    @pl.loop(jnp.int32(0), jnp.int32(CPW), step=2)
    def chunk_body(ci):
        fire_in(ci + 1, 1)

        @pl.when(ci > 0)
        def _():
            drain_out(0)
        drain_in(0)
        process(0)
        fire_out(ci, 0)

        @pl.when(ci + 2 < CPW)
        def _():
            fire_in(ci + 2, 0)

        @pl.when(ci > 0)
        def _():
            drain_out(1)
        drain_in(1)
        process(1)
        fire_out(ci + 1, 1)

    drain_out(0)
    drain_out(1)


@functools.cache
def _build_fmt_kernel():
    return pl.kernel(
        _fmt_body,
        out_type=jax.ShapeDtypeStruct((TABLE_SIZE * DIM,), jnp.float32),
        mesh=plsc.VectorSubcoreMesh(core_axis_name="c", subcore_axis_name="s",
                                    num_cores=NC, num_subcores=NS),
        scratch_types=[
            pltpu.VMEM((2, CW), jnp.float32),
            pltpu.VMEM((2, CW), jnp.float32),
            pltpu.SemaphoreType.DMA((2,)),
            pltpu.SemaphoreType.DMA((2,)),
        ],
        **_SC_PARAMS,
    )


def _sc_body(table_h, hist_h, inp_h, nodes_h, out_h,
             jall_v, j_v, tidx_v, node_v, feat_v, inp_v, out_v, gsem):
    wid = lax.axis_index("s") * NC + lax.axis_index("c")
    iota = lax.iota(jnp.int32, 16)
    laneconst = (iota >> 3) * 2048 + (iota & 7) * 128

    def build_and_fire(i, slot):
        @pl.loop(jnp.int32(0), jnp.int32(NG))
        def tidx_body(g):
            lane = g * 16 + iota
            jv = plsc.load_gather(jall_v, [_splat(i), lane])
            j_v[jnp.int32(slot), pl.ds(g * 16, 16)] = jv
            for c in range(8):
                tidx_v[jnp.int32(slot), pl.ds(c * B_R + g * 16, 16)] = (
                    jv ^ jnp.int32(K_XOR[c]))

        pltpu.async_copy(nodes_h.at[j_v.at[jnp.int32(slot)]],
                         node_v.at[jnp.int32(slot)], gsem.at[jnp.int32(slot)])
        pltpu.async_copy(table_h.at[tidx_v.at[jnp.int32(slot)]],
                         feat_v.at[jnp.int32(slot)], gsem.at[jnp.int32(slot)])

    def drain(slot):
        pltpu.make_async_copy(nodes_h.at[j_v.at[jnp.int32(slot)]],
                              node_v.at[jnp.int32(slot)],
                              gsem.at[jnp.int32(slot)]).wait()
        pltpu.make_async_copy(table_h.at[tidx_v.at[jnp.int32(slot)]],
                              feat_v.at[jnp.int32(slot)],
                              gsem.at[jnp.int32(slot)]).wait()

    def compute(i, slot):
        ihi = i >> 2
        ilo = i & 3
        colbase = ihi * 1024 + ilo * 32

        @pl.loop(jnp.int32(0), jnp.int32(NG))
        def group_body(g):
            lane = g * 16 + iota
            laneterm = laneconst + g * 4096
            nm = [plsc.load_gather(node_v, [_splat(slot), lane, _splat(comp)])
                  for comp in range(3)]
            ie = [1.0 / plsc.load_gather(node_v,
                                         [_splat(slot), lane, _splat(3 + comp)])
                  for comp in range(3)]
            w = []
            for k in range(NPTS):
                px = jnp.clip((inp_v[jnp.int32(3 * k + 0), pl.ds(g * 16, 16)] - nm[0]) * ie[0], 0.0, 1.0)
                py = jnp.clip((inp_v[jnp.int32(3 * k + 1), pl.ds(g * 16, 16)] - nm[1]) * ie[1], 0.0, 1.0)
                pz = jnp.clip((inp_v[jnp.int32(3 * k + 2), pl.ds(g * 16, 16)] - nm[2]) * ie[2], 0.0, 1.0)
                ax, ay, az = 1.0 - px, 1.0 - py, 1.0 - pz
                b00, b10, b01, b11 = ax * ay, px * ay, ax * py, px * py
                w.append((b00 * az, b10 * az, b01 * az, b00 * pz,
                          b10 * pz, b01 * pz, b11 * az, b11 * pz))
            for d in range(DIM):
                f = [plsc.load_gather(
                    feat_v, [_splat(slot), _splat(c * B_R) + lane, _splat(d)])
                    for c in range(8)]
                for k in range(NPTS):
                    acc = w[k][0] * f[0]
                    for c in range(1, 8):
                        acc = acc + w[k][c] * f[c]
                    pos = laneterm + (colbase + (k * 8 + d))
                    plsc.store_scatter(out_v, [pos], acc)

    @pl.loop(jnp.int32(0), jnp.int32(NBLK))
    def block_body(b):
        base = wid * RPW + b * B_R
        pltpu.sync_copy(hist_h.at[:, pl.ds(base, B_R)], jall_v)
        pltpu.sync_copy(inp_h.at[:, pl.ds(base, B_R)], inp_v)

        build_and_fire(jnp.int32(0), 0)

        @pl.loop(jnp.int32(0), jnp.int32(DEPTH), step=2)
        def depth_body(i):
            build_and_fire(i + 1, 1)
            drain(0)
            compute(i, 0)

            @pl.when(i + 2 < DEPTH)
            def _():
                build_and_fire(i + 2, 0)

            drain(1)
            compute(i + 1, 1)

        pltpu.sync_copy(out_v, out_h.at[pl.ds(base * 256, B_R * 256)])


@functools.cache
def _build_sc_kernel():
    return pl.kernel(
        _sc_body,
        out_type=jax.ShapeDtypeStruct((R * DEPTH * NPTS * DIM,), jnp.float32),
        mesh=plsc.VectorSubcoreMesh(core_axis_name="c", subcore_axis_name="s",
                                    num_cores=NC, num_subcores=NS),
        scratch_types=[
            pltpu.VMEM((DEPTH, B_R), jnp.int32),
            pltpu.VMEM((2, B_R), jnp.int32),
            pltpu.VMEM((2, 8 * B_R), jnp.int32),
            pltpu.VMEM((2, B_R, 8), jnp.float32),
            pltpu.VMEM((2, 8 * B_R, DIM), jnp.float32),
            pltpu.VMEM((3 * NPTS, B_R), jnp.float32),
            pltpu.VMEM((B_R * DEPTH * NPTS * DIM,), jnp.float32),
            pltpu.SemaphoreType.DMA((2,)),
        ],
        **_SC_PARAMS,
    )


def kernel(inp, history, table, nodes_min, nodes_extent):
    table_t = (table.reshape(NGRP, 128, DIM).transpose(0, 2, 1)
               .reshape(NGRP * DIM * 128))
    table_lin = _build_fmt_kernel()(table_t).reshape(TABLE_SIZE, DIM)

    hist_t = history.astype(jnp.int32).T
    inp_t = inp.astype(jnp.float32).reshape(R, 3 * NPTS).T
    nodes_cat = jnp.concatenate(
        [nodes_min.astype(jnp.float32), nodes_extent.astype(jnp.float32),
         jnp.zeros((nodes_min.shape[0], 2), jnp.float32)], axis=1)
    raw = _build_sc_kernel()(table_lin, hist_t, inp_t, nodes_cat)
    return (raw.reshape(R // 8, 2, 8, 128).transpose(0, 2, 1, 3)
            .reshape(R, DEPTH * NPTS * DIM))
